# baseline (speedup 1.0000x reference)
import functools

import jax
import jax.numpy as jnp
from jax import lax
from jax.experimental import pallas as pl
from jax.experimental.pallas import tpu as pltpu
from jax.experimental.pallas import tpu_sc as plsc

N = 27648
E = 442368
B = 512
NPG = 54
NC = 2
NS = 16
CH = 128
ZR = 216
RPT = N // NS

_f32 = jnp.float32
_i32 = jnp.int32


def _mm(a, b):
    return jnp.dot(a.astype(jnp.bfloat16), b.astype(jnp.bfloat16),
                   preferred_element_type=_f32)


def _tcmid_body(agg_ref, x0_ref, x1_ref, x2_ref, x3_ref,
                wrel_ref, wroot_ref, b1_ref,
                h0_ref, h1_ref, h2_ref, h3_ref):
    agg = jnp.concatenate([agg_ref[0], agg_ref[1], agg_ref[2], agg_ref[3]],
                          axis=1)
    xc = jnp.concatenate([x0_ref[...], x1_ref[...], x2_ref[...], x3_ref[...]],
                         axis=1)
    h = jnp.maximum(_mm(agg, wrel_ref[...]) + b1_ref[...]
                    + _mm(xc, wroot_ref[...]), 0.0)
    h0_ref[...] = h[:, 0:32]
    h1_ref[...] = h[:, 32:64]
    h2_ref[...] = h[:, 64:96]
    h3_ref[...] = h[:, 96:128]


def _tcmid(agg, x0, x1, x2, x3, wrel, wroot, b1):
    blk = 1024
    grid = (N // blk,)
    qspec = pl.BlockSpec((blk, 32), lambda i: (i, 0))
    return pl.pallas_call(
        _tcmid_body,
        grid=grid,
        in_specs=[
            pl.BlockSpec((4, blk, 32), lambda i: (0, i, 0)),
            qspec, qspec, qspec, qspec,
            pl.BlockSpec((128, 128), lambda i: (0, 0)),
            pl.BlockSpec((128, 128), lambda i: (0, 0)),
            pl.BlockSpec((1, 128), lambda i: (0, 0)),
        ],
        out_specs=[qspec, qspec, qspec, qspec],
        out_shape=[jax.ShapeDtypeStruct((N, 32), _f32)] * 4,
    )(agg, x0, x1, x2, x3, wrel, wroot, b1)


NB = 3
SB = 108


def _zero_acc(zbuf, acc_sh, s, qw):
    def _zrow(r, carry):
        for q in range(qw):
            zbuf[r, pl.ds(q * 16, 16)] = jnp.zeros((16,), _f32)
        return carry
    lax.fori_loop(0, ZR, _zrow, 0)

    def _zcp(i, carry):
        pltpu.sync_copy(zbuf, acc_sh.at[pl.ds(s * RPT + i * ZR, ZR)])
        return carry
    lax.fori_loop(0, RPT // ZR, _zcp, 0)


def _edge_superchunk(row0, tbl_hbm, src2d, dst2d, w2d,
                     src_sv, dst_sv, w_sv, rows, sbuf, acc_sh,
                     gsems, ssems, qw):
    pltpu.sync_copy(src2d.at[pl.ds(row0, SB)], src_sv)
    pltpu.sync_copy(dst2d.at[pl.ds(row0, SB)], dst_sv)
    pltpu.sync_copy(w2d.at[pl.ds(row0, SB)], w_sv)
    for b in range(NB):
        pltpu.async_copy(tbl_hbm.at[src_sv.at[b]], rows[b], gsems[b])

    def _round(r, carry):
        scats = []
        for b in range(NB):
            jl = r * NB + b
            pltpu.make_async_copy(tbl_hbm.at[src_sv.at[jl]],
                                  rows[b], gsems[b]).wait()

            def _grp(g, cc):
                jsplat = jnp.full((16,), jl, _i32)
                wbs = [plsc.load_gather(
                    w_sv, (jsplat, jnp.full((16,), g * 16 + u, _i32)))
                    for u in range(16)]
                for u in range(16):
                    e = g * 16 + u
                    for q in range(qw):
                        sl = pl.ds(q * 16, 16)
                        sbuf[b][e, sl] = rows[b][e, sl] * wbs[u]
                return cc
            lax.fori_loop(0, CH // 16, _grp, 0)

            @pl.when(jl + NB < SB)
            def _():
                pltpu.async_copy(tbl_hbm.at[src_sv.at[jl + NB]],
                                 rows[b], gsems[b])
            scats.append(pltpu.async_copy(sbuf[b], acc_sh.at[dst_sv.at[jl]],
                                          ssems[b], add=True))
        for cp in scats:
            cp.wait()
        return carry
    lax.fori_loop(0, SB // NB, _round, 0)


def _sc1_body(src2d, dst2d, w2d, t0, t1, t2, t3, agg_hbm,
              src_sv, dst_sv, w_sv,
              r0, r1, r2, s0, s1, s2, zbuf, acc_sh,
              g0, g1, g2, q0, q1, q2):
    c = lax.axis_index("c")
    s = lax.axis_index("s")
    rows, sbuf = [r0, r1, r2], [s0, s1, s2]
    gsems, ssems = [g0, g1, g2], [q0, q1, q2]

    rpt = (E // CH) // NS

    for p in range(2):
        _zero_acc(zbuf, acc_sh, s, 2)
        plsc.subcore_barrier()

        def _run(tbl_hbm):
            def _sc(k, carry):
                _edge_superchunk(s * rpt + k * SB, tbl_hbm,
                                 src2d, dst2d, w2d,
                                 src_sv, dst_sv, w_sv, rows, sbuf, acc_sh,
                                 gsems, ssems, 2)
                return carry
            lax.fori_loop(0, rpt // SB, _sc, 0)

        @pl.when(c == 0)
        def _():
            _run((t0, t1)[p])

        @pl.when(c == 1)
        def _():
            _run((t2, t3)[p])

        plsc.subcore_barrier()

        def _wb(i, carry):
            rr = s * RPT + i * ZR
            pltpu.sync_copy(acc_sh.at[pl.ds(rr, ZR)],
                            agg_hbm.at[c * 2 + p, pl.ds(rr, ZR)])
            return carry
        lax.fori_loop(0, RPT // ZR, _wb, 0)


def _sc1(src2d, dst2d, w2d, t0, t1, t2, t3):
    f = pl.kernel(
        _sc1_body,
        out_type=jax.ShapeDtypeStruct((4, N, 32), _f32),
        compiler_params=pltpu.CompilerParams(needs_layout_passes=False, use_tc_tiling_on_sc=False),
        mesh=plsc.VectorSubcoreMesh(core_axis_name="c", subcore_axis_name="s"),
        scratch_types=[
            pltpu.VMEM((SB, CH), _i32),
            pltpu.VMEM((SB, CH), _i32),
            pltpu.VMEM((SB, CH), _f32),
            pltpu.VMEM((CH, 32), _f32),
            pltpu.VMEM((CH, 32), _f32),
            pltpu.VMEM((CH, 32), _f32),
            pltpu.VMEM((CH, 32), _f32),
            pltpu.VMEM((CH, 32), _f32),
            pltpu.VMEM((CH, 32), _f32),
            pltpu.VMEM((ZR, 32), _f32),
            pltpu.VMEM_SHARED((N, 32), _f32),
            pltpu.SemaphoreType.DMA,
            pltpu.SemaphoreType.DMA,
            pltpu.SemaphoreType.DMA,
            pltpu.SemaphoreType.DMA,
            pltpu.SemaphoreType.DMA,
            pltpu.SemaphoreType.DMA,
        ],
    )
    return f(src2d, dst2d, w2d, t0, t1, t2, t3)


def _tc2_body(agg_ref, h0_ref, h1_ref, h2_ref, h3_ref,
              w2p_ref, wr2p_ref, b2p_ref, h2p_ref):
    agg = jnp.concatenate([agg_ref[0], agg_ref[1], agg_ref[2], agg_ref[3]],
                          axis=1)
    hc = jnp.concatenate([h0_ref[...], h1_ref[...], h2_ref[...], h3_ref[...]],
                         axis=1)
    h2p_ref[...] = jnp.maximum(_mm(agg, w2p_ref[...]) + b2p_ref[...]
                               + _mm(hc, wr2p_ref[...]), 0.0)


def _tc2(agg, h0, h1, h2, h3, w2p, wr2p, b2p):
    blk = 1024
    grid = (N // blk,)
    qspec = pl.BlockSpec((blk, 32), lambda i: (i, 0))
    return pl.pallas_call(
        _tc2_body,
        grid=grid,
        in_specs=[
            pl.BlockSpec((4, blk, 32), lambda i: (0, i, 0)),
            qspec, qspec, qspec, qspec,
            pl.BlockSpec((128, 16), lambda i: (0, 0)),
            pl.BlockSpec((128, 16), lambda i: (0, 0)),
            pl.BlockSpec((1, 16), lambda i: (0, 0)),
        ],
        out_specs=pl.BlockSpec((blk, 16), lambda i: (i, 0)),
        out_shape=jax.ShapeDtypeStruct((N, 16), _f32),
    )(agg, h0, h1, h2, h3, w2p, wr2p, b2p)


def _head_body(h2g_ref, scl_ref, sclg_ref, gf_ref,
               wg1_ref, bg1_ref, wg2_ref, bg2_ref, wg3_ref, bg3_ref,
               wo1a_ref, wo1b_ref, bo1_ref, wo2_ref, bo2_ref, out_ref):
    h2 = h2g_ref[...] * scl_ref[...]
    g = jnp.maximum(_mm(gf_ref[...], wg1_ref[...]) + bg1_ref[...], 0.0)
    g = jnp.maximum(_mm(g, wg2_ref[...]) + bg2_ref[...], 0.0)
    g = jnp.maximum(_mm(g, wg3_ref[...]) + bg3_ref[...], 0.0)
    g = g * sclg_ref[...]
    o1 = jnp.maximum(_mm(h2, wo1a_ref[...]) + _mm(g, wo1b_ref[...])
                     + bo1_ref[...], 0.0)
    o2 = _mm(o1, wo2_ref[...]) + bo2_ref[...]
    out_ref[...] = jax.nn.sigmoid(o2)


def _head(h2g, scl, sclg, gf,
          wg1, bg1, wg2, bg2, wg3, bg3, wo1a, wo1b, bo1, wo2, bo2):
    return pl.pallas_call(
        _head_body,
        out_shape=jax.ShapeDtypeStruct((B, 1), _f32),
    )(h2g, scl, sclg, gf,
      wg1, bg1, wg2, bg2, wg3, bg3, wo1a, wo1b, bo1, wo2, bo2)


def kernel(x, edge_index, edge_attr, globalFeats, isTrain,
           W_rel1, b_rel1, W_root1, W_rel2, b_rel2, W_root2,
           Wg1, bg1, Wg2, bg2, Wg3, bg3, Wo1, bo1, Wo2, bo2):
    src = jnp.asarray(edge_index[0], _i32).reshape(E // CH, CH)
    dst = jnp.asarray(edge_index[1], _i32).reshape(E // CH, CH)
    w = jnp.asarray(edge_attr, _f32).reshape(E // CH, CH)

    xq = [x[:, 32 * q:32 * (q + 1)] for q in range(4)]
    agg1 = _sc1(src, dst, w, *xq)
    h0, h1, h2, h3 = _tcmid(agg1, *xq, W_rel1, W_root1, b_rel1.reshape(1, 128))

    agg2 = _sc1(src, dst, w, h0, h1, h2, h3)
    pad = ((0, 0), (0, 12))
    w2p = jnp.pad(W_rel2, pad)
    wr2p = jnp.pad(W_root2, pad)
    b2p = jnp.pad(b_rel2, (0, 12)).reshape(1, 16)
    h2p = _tc2(agg2, h0, h1, h2, h3, w2p, wr2p, b2p)

    h2g = h2p.reshape(B, NPG * 16)

    d_cat = NPG * 4 + 16
    keep = jax.random.bernoulli(jax.random.key(42), 0.8, (B, d_cat))
    scale = jnp.where(jnp.asarray(isTrain),
                      jnp.where(keep, 1.0 / 0.8, 0.0),
                      1.0).astype(_f32)
    scl_emb = jnp.pad(scale[:, :NPG * 4].reshape(B, NPG, 4),
                      ((0, 0), (0, 0), (0, 12))).reshape(B, NPG * 16)
    scl_g = scale[:, NPG * 4:]

    wo1a = jnp.pad(Wo1[:NPG * 4].reshape(NPG, 4, 128),
                   ((0, 0), (0, 12), (0, 0))).reshape(NPG * 16, 128)
    wo1b = Wo1[NPG * 4:]

    return _head(h2g, scl_emb, scl_g, globalFeats,
                 Wg1, bg1.reshape(1, 8), Wg2, bg2.reshape(1, 8),
                 Wg3, bg3.reshape(1, 16), wo1a, wo1b, bo1.reshape(1, 128),
                 Wo2, bo2.reshape(1, 1))


# speedup vs baseline: 1.1431x; 1.1431x over previous
import functools

import jax
import jax.numpy as jnp
from jax import lax
from jax.experimental import pallas as pl
from jax.experimental.pallas import tpu as pltpu
from jax.experimental.pallas import tpu_sc as plsc

N = 27648
E = 442368
B = 512
NPG = 54
NC = 2
NS = 16
CH = 128
ZR = 216
RPT = N // NS

_f32 = jnp.float32
_i32 = jnp.int32


def _mm(a, b):
    return jnp.dot(a.astype(jnp.bfloat16), b.astype(jnp.bfloat16),
                   preferred_element_type=_f32)


def _tcmid_body(agg_ref, x0_ref, x1_ref, x2_ref, x3_ref,
                wrel_ref, wroot_ref, b1_ref,
                h0_ref, h1_ref, h2_ref, h3_ref):
    agg = jnp.concatenate([agg_ref[0], agg_ref[1], agg_ref[2], agg_ref[3]],
                          axis=1)
    xc = jnp.concatenate([x0_ref[...], x1_ref[...], x2_ref[...], x3_ref[...]],
                         axis=1)
    h = jnp.maximum(_mm(agg, wrel_ref[...]) + b1_ref[...]
                    + _mm(xc, wroot_ref[...]), 0.0)
    h0_ref[...] = h[:, 0:32]
    h1_ref[...] = h[:, 32:64]
    h2_ref[...] = h[:, 64:96]
    h3_ref[...] = h[:, 96:128]


def _tcmid(agg, x0, x1, x2, x3, wrel, wroot, b1):
    blk = 1024
    grid = (N // blk,)
    qspec = pl.BlockSpec((blk, 32), lambda i: (i, 0))
    return pl.pallas_call(
        _tcmid_body,
        grid=grid,
        in_specs=[
            pl.BlockSpec((4, blk, 32), lambda i: (0, i, 0)),
            qspec, qspec, qspec, qspec,
            pl.BlockSpec((128, 128), lambda i: (0, 0)),
            pl.BlockSpec((128, 128), lambda i: (0, 0)),
            pl.BlockSpec((1, 128), lambda i: (0, 0)),
        ],
        out_specs=[qspec, qspec, qspec, qspec],
        out_shape=[jax.ShapeDtypeStruct((N, 32), _f32)] * 4,
    )(agg, x0, x1, x2, x3, wrel, wroot, b1)


NB = 3
SB = 108


def _zero_acc(zbuf, acc_sh, s, qw):
    def _zrow(r, carry):
        for q in range(qw):
            zbuf[r, pl.ds(q * 16, 16)] = jnp.zeros((16,), _f32)
        return carry
    lax.fori_loop(0, ZR, _zrow, 0)

    def _zcp(i, carry):
        pltpu.sync_copy(zbuf, acc_sh.at[pl.ds(s * RPT + i * ZR, ZR)])
        return carry
    lax.fori_loop(0, RPT // ZR, _zcp, 0)


def _edge_superchunk(row0, tbl_hbm, src2d, dst2d, wrep_hbm,
                     src_sv, dst_sv, rows, wreps, sbuf, acc_sh,
                     gsems, ssems, qw):
    pltpu.sync_copy(src2d.at[pl.ds(row0, SB)], src_sv)
    pltpu.sync_copy(dst2d.at[pl.ds(row0, SB)], dst_sv)
    for b in range(NB):
        pltpu.async_copy(tbl_hbm.at[src_sv.at[b]], rows[b], gsems[b])
        pltpu.async_copy(wrep_hbm.at[pl.ds((row0 + b) * CH, CH)],
                         wreps[b], gsems[b])

    def _round(r, carry):
        scats = []
        for b in range(NB):
            jl = r * NB + b
            pltpu.make_async_copy(tbl_hbm.at[src_sv.at[jl]],
                                  rows[b], gsems[b]).wait()
            pltpu.make_async_copy(wrep_hbm.at[pl.ds((row0 + jl) * CH, CH)],
                                  wreps[b], gsems[b]).wait()

            def _grp(g, cc):
                for u in range(16):
                    e = g * 16 + u
                    wb = wreps[b][e, :]
                    for q in range(qw):
                        sl = pl.ds(q * 16, 16)
                        sbuf[b][e, sl] = rows[b][e, sl] * wb
                return cc
            lax.fori_loop(0, CH // 16, _grp, 0)

            @pl.when(jl + NB < SB)
            def _():
                pltpu.async_copy(tbl_hbm.at[src_sv.at[jl + NB]],
                                 rows[b], gsems[b])
                pltpu.async_copy(wrep_hbm.at[pl.ds((row0 + jl + NB) * CH, CH)],
                                 wreps[b], gsems[b])
            scats.append(pltpu.async_copy(sbuf[b], acc_sh.at[dst_sv.at[jl]],
                                          ssems[b], add=True))
        for cp in scats:
            cp.wait()
        return carry
    lax.fori_loop(0, SB // NB, _round, 0)


def _sc1_body(src2d, dst2d, wrep_hbm, t0, t1, t2, t3, agg_hbm,
              src_sv, dst_sv,
              r0, r1, r2, w0, w1, w2, s0, s1, s2, zbuf, acc_sh,
              g0, g1, g2, q0, q1, q2):
    c = lax.axis_index("c")
    s = lax.axis_index("s")
    rows, wreps, sbuf = [r0, r1, r2], [w0, w1, w2], [s0, s1, s2]
    gsems, ssems = [g0, g1, g2], [q0, q1, q2]

    rpt = (E // CH) // NS

    for p in range(2):
        _zero_acc(zbuf, acc_sh, s, 2)
        plsc.subcore_barrier()

        def _run(tbl_hbm):
            def _sc(k, carry):
                _edge_superchunk(s * rpt + k * SB, tbl_hbm,
                                 src2d, dst2d, wrep_hbm,
                                 src_sv, dst_sv, rows, wreps, sbuf, acc_sh,
                                 gsems, ssems, 2)
                return carry
            lax.fori_loop(0, rpt // SB, _sc, 0)

        @pl.when(c == 0)
        def _():
            _run((t0, t1)[p])

        @pl.when(c == 1)
        def _():
            _run((t2, t3)[p])

        plsc.subcore_barrier()

        def _wb(i, carry):
            rr = s * RPT + i * ZR
            pltpu.sync_copy(acc_sh.at[pl.ds(rr, ZR)],
                            agg_hbm.at[c * 2 + p, pl.ds(rr, ZR)])
            return carry
        lax.fori_loop(0, RPT // ZR, _wb, 0)


def _sc1(src2d, dst2d, wrep, t0, t1, t2, t3):
    f = pl.kernel(
        _sc1_body,
        out_type=jax.ShapeDtypeStruct((4, N, 32), _f32),
        compiler_params=pltpu.CompilerParams(needs_layout_passes=False, use_tc_tiling_on_sc=False),
        mesh=plsc.VectorSubcoreMesh(core_axis_name="c", subcore_axis_name="s"),
        scratch_types=[
            pltpu.VMEM((SB, CH), _i32),
            pltpu.VMEM((SB, CH), _i32),
            pltpu.VMEM((CH, 32), _f32),
            pltpu.VMEM((CH, 32), _f32),
            pltpu.VMEM((CH, 32), _f32),
            pltpu.VMEM((CH, 16), _f32),
            pltpu.VMEM((CH, 16), _f32),
            pltpu.VMEM((CH, 16), _f32),
            pltpu.VMEM((CH, 32), _f32),
            pltpu.VMEM((CH, 32), _f32),
            pltpu.VMEM((CH, 32), _f32),
            pltpu.VMEM((ZR, 32), _f32),
            pltpu.VMEM_SHARED((N, 32), _f32),
            pltpu.SemaphoreType.DMA,
            pltpu.SemaphoreType.DMA,
            pltpu.SemaphoreType.DMA,
            pltpu.SemaphoreType.DMA,
            pltpu.SemaphoreType.DMA,
            pltpu.SemaphoreType.DMA,
        ],
    )
    return f(src2d, dst2d, wrep, t0, t1, t2, t3)


def _tc2_body(agg_ref, h0_ref, h1_ref, h2_ref, h3_ref,
              w2p_ref, wr2p_ref, b2p_ref, h2p_ref):
    agg = jnp.concatenate([agg_ref[0], agg_ref[1], agg_ref[2], agg_ref[3]],
                          axis=1)
    hc = jnp.concatenate([h0_ref[...], h1_ref[...], h2_ref[...], h3_ref[...]],
                         axis=1)
    h2p_ref[...] = jnp.maximum(_mm(agg, w2p_ref[...]) + b2p_ref[...]
                               + _mm(hc, wr2p_ref[...]), 0.0)


def _tc2(agg, h0, h1, h2, h3, w2p, wr2p, b2p):
    blk = 1024
    grid = (N // blk,)
    qspec = pl.BlockSpec((blk, 32), lambda i: (i, 0))
    return pl.pallas_call(
        _tc2_body,
        grid=grid,
        in_specs=[
            pl.BlockSpec((4, blk, 32), lambda i: (0, i, 0)),
            qspec, qspec, qspec, qspec,
            pl.BlockSpec((128, 16), lambda i: (0, 0)),
            pl.BlockSpec((128, 16), lambda i: (0, 0)),
            pl.BlockSpec((1, 16), lambda i: (0, 0)),
        ],
        out_specs=pl.BlockSpec((blk, 16), lambda i: (i, 0)),
        out_shape=jax.ShapeDtypeStruct((N, 16), _f32),
    )(agg, h0, h1, h2, h3, w2p, wr2p, b2p)


def _head_body(h2g_ref, scl_ref, sclg_ref, gf_ref,
               wg1_ref, bg1_ref, wg2_ref, bg2_ref, wg3_ref, bg3_ref,
               wo1a_ref, wo1b_ref, bo1_ref, wo2_ref, bo2_ref, out_ref):
    h2 = h2g_ref[...] * scl_ref[...]
    g = jnp.maximum(_mm(gf_ref[...], wg1_ref[...]) + bg1_ref[...], 0.0)
    g = jnp.maximum(_mm(g, wg2_ref[...]) + bg2_ref[...], 0.0)
    g = jnp.maximum(_mm(g, wg3_ref[...]) + bg3_ref[...], 0.0)
    g = g * sclg_ref[...]
    o1 = jnp.maximum(_mm(h2, wo1a_ref[...]) + _mm(g, wo1b_ref[...])
                     + bo1_ref[...], 0.0)
    o2 = _mm(o1, wo2_ref[...]) + bo2_ref[...]
    out_ref[...] = jax.nn.sigmoid(o2)


def _head(h2g, scl, sclg, gf,
          wg1, bg1, wg2, bg2, wg3, bg3, wo1a, wo1b, bo1, wo2, bo2):
    return pl.pallas_call(
        _head_body,
        out_shape=jax.ShapeDtypeStruct((B, 1), _f32),
    )(h2g, scl, sclg, gf,
      wg1, bg1, wg2, bg2, wg3, bg3, wo1a, wo1b, bo1, wo2, bo2)


def kernel(x, edge_index, edge_attr, globalFeats, isTrain,
           W_rel1, b_rel1, W_root1, W_rel2, b_rel2, W_root2,
           Wg1, bg1, Wg2, bg2, Wg3, bg3, Wo1, bo1, Wo2, bo2):
    src = jnp.asarray(edge_index[0], _i32).reshape(E // CH, CH)
    dst = jnp.asarray(edge_index[1], _i32).reshape(E // CH, CH)
    wrep = jnp.broadcast_to(jnp.asarray(edge_attr, _f32)[:, None], (E, 16))

    xq = [x[:, 32 * q:32 * (q + 1)] for q in range(4)]
    agg1 = _sc1(src, dst, wrep, *xq)
    h0, h1, h2, h3 = _tcmid(agg1, *xq, W_rel1, W_root1, b_rel1.reshape(1, 128))

    agg2 = _sc1(src, dst, wrep, h0, h1, h2, h3)
    pad = ((0, 0), (0, 12))
    w2p = jnp.pad(W_rel2, pad)
    wr2p = jnp.pad(W_root2, pad)
    b2p = jnp.pad(b_rel2, (0, 12)).reshape(1, 16)
    h2p = _tc2(agg2, h0, h1, h2, h3, w2p, wr2p, b2p)

    h2g = h2p.reshape(B, NPG * 16)

    d_cat = NPG * 4 + 16
    keep = jax.random.bernoulli(jax.random.key(42), 0.8, (B, d_cat))
    scale = jnp.where(jnp.asarray(isTrain),
                      jnp.where(keep, 1.0 / 0.8, 0.0),
                      1.0).astype(_f32)
    scl_emb = jnp.pad(scale[:, :NPG * 4].reshape(B, NPG, 4),
                      ((0, 0), (0, 0), (0, 12))).reshape(B, NPG * 16)
    scl_g = scale[:, NPG * 4:]

    wo1a = jnp.pad(Wo1[:NPG * 4].reshape(NPG, 4, 128),
                   ((0, 0), (0, 12), (0, 0))).reshape(NPG * 16, 128)
    wo1b = Wo1[NPG * 4:]

    return _head(h2g, scl_emb, scl_g, globalFeats,
                 Wg1, bg1.reshape(1, 8), Wg2, bg2.reshape(1, 8),
                 Wg3, bg3.reshape(1, 16), wo1a, wo1b, bo1.reshape(1, 128),
                 Wo2, bo2.reshape(1, 1))


# speedup vs baseline: 1.1518x; 1.0075x over previous
import functools

import jax
import jax.numpy as jnp
from jax import lax
from jax.experimental import pallas as pl
from jax.experimental.pallas import tpu as pltpu
from jax.experimental.pallas import tpu_sc as plsc

N = 27648
E = 442368
B = 512
NPG = 54
NC = 2
NS = 16
CH = 128
ZR = 216
RPT = N // NS

_f32 = jnp.float32
_i32 = jnp.int32


def _mm(a, b):
    return jnp.dot(a.astype(jnp.bfloat16), b.astype(jnp.bfloat16),
                   preferred_element_type=_f32)


def _tcmid_body(agg_ref, x0_ref, x1_ref, x2_ref, x3_ref,
                wrel_ref, wroot_ref, b1_ref,
                h0_ref, h1_ref, h2_ref, h3_ref):
    agg = jnp.concatenate([agg_ref[0], agg_ref[1], agg_ref[2], agg_ref[3]],
                          axis=1)
    xc = jnp.concatenate([x0_ref[...], x1_ref[...], x2_ref[...], x3_ref[...]],
                         axis=1)
    h = jnp.maximum(_mm(agg, wrel_ref[...]) + b1_ref[...]
                    + _mm(xc, wroot_ref[...]), 0.0)
    h0_ref[...] = h[:, 0:32]
    h1_ref[...] = h[:, 32:64]
    h2_ref[...] = h[:, 64:96]
    h3_ref[...] = h[:, 96:128]


def _tcmid(agg, x0, x1, x2, x3, wrel, wroot, b1):
    blk = 1024
    grid = (N // blk,)
    qspec = pl.BlockSpec((blk, 32), lambda i: (i, 0))
    return pl.pallas_call(
        _tcmid_body,
        grid=grid,
        in_specs=[
            pl.BlockSpec((4, blk, 32), lambda i: (0, i, 0)),
            qspec, qspec, qspec, qspec,
            pl.BlockSpec((128, 128), lambda i: (0, 0)),
            pl.BlockSpec((128, 128), lambda i: (0, 0)),
            pl.BlockSpec((1, 128), lambda i: (0, 0)),
        ],
        out_specs=[qspec, qspec, qspec, qspec],
        out_shape=[jax.ShapeDtypeStruct((N, 32), _f32)] * 4,
    )(agg, x0, x1, x2, x3, wrel, wroot, b1)


NB = 4
SB = 108


def _zero_acc(zbuf, acc_sh, s, qw):
    def _zrow(r, carry):
        for q in range(qw):
            zbuf[r, pl.ds(q * 16, 16)] = jnp.zeros((16,), _f32)
        return carry
    lax.fori_loop(0, ZR, _zrow, 0)

    def _zcp(i, carry):
        pltpu.sync_copy(zbuf, acc_sh.at[pl.ds(s * RPT + i * ZR, ZR)])
        return carry
    lax.fori_loop(0, RPT // ZR, _zcp, 0)


def _edge_superchunk(row0, tbl_hbm, src2d, dst2d, wrep_hbm,
                     src_sv, dst_sv, rows, wreps, sbuf, acc_sh,
                     gsems, ssems, qw):
    pltpu.sync_copy(src2d.at[pl.ds(row0, SB)], src_sv)
    pltpu.sync_copy(dst2d.at[pl.ds(row0, SB)], dst_sv)
    for b in range(NB):
        pltpu.async_copy(tbl_hbm.at[src_sv.at[b]], rows[b], gsems[b])
        pltpu.async_copy(wrep_hbm.at[pl.ds((row0 + b) * CH, CH)],
                         wreps[b], gsems[b])

    def _round(r, carry):
        scats = []
        for b in range(NB):
            jl = r * NB + b
            pltpu.make_async_copy(tbl_hbm.at[src_sv.at[jl]],
                                  rows[b], gsems[b]).wait()
            pltpu.make_async_copy(wrep_hbm.at[pl.ds((row0 + jl) * CH, CH)],
                                  wreps[b], gsems[b]).wait()

            def _grp(g, cc):
                for u in range(16):
                    e = g * 16 + u
                    wb = wreps[b][e, :]
                    for q in range(qw):
                        sl = pl.ds(q * 16, 16)
                        sbuf[b][e, sl] = rows[b][e, sl] * wb
                return cc
            lax.fori_loop(0, CH // 16, _grp, 0)

            @pl.when(jl + NB < SB)
            def _():
                pltpu.async_copy(tbl_hbm.at[src_sv.at[jl + NB]],
                                 rows[b], gsems[b])
                pltpu.async_copy(wrep_hbm.at[pl.ds((row0 + jl + NB) * CH, CH)],
                                 wreps[b], gsems[b])
            scats.append(pltpu.async_copy(sbuf[b], acc_sh.at[dst_sv.at[jl]],
                                          ssems[b], add=True))
        for cp in scats:
            cp.wait()
        return carry
    lax.fori_loop(0, SB // NB, _round, 0)


def _sc1_body(src2d, dst2d, wrep_hbm, t0, t1, t2, t3, agg_hbm,
              src_sv, dst_sv,
              r0, r1, r2, r3, w0, w1, w2, w3, s0, s1, s2, s3, zbuf, acc_sh,
              g0, g1, g2, g3, q0, q1, q2, q3):
    c = lax.axis_index("c")
    s = lax.axis_index("s")
    rows, wreps = [r0, r1, r2, r3], [w0, w1, w2, w3]
    sbuf = [s0, s1, s2, s3]
    gsems, ssems = [g0, g1, g2, g3], [q0, q1, q2, q3]

    rpt = (E // CH) // NS

    for p in range(2):
        _zero_acc(zbuf, acc_sh, s, 2)
        plsc.subcore_barrier()

        def _run(tbl_hbm):
            def _sc(k, carry):
                _edge_superchunk(s * rpt + k * SB, tbl_hbm,
                                 src2d, dst2d, wrep_hbm,
                                 src_sv, dst_sv, rows, wreps, sbuf, acc_sh,
                                 gsems, ssems, 2)
                return carry
            lax.fori_loop(0, rpt // SB, _sc, 0)

        @pl.when(c == 0)
        def _():
            _run((t0, t1)[p])

        @pl.when(c == 1)
        def _():
            _run((t2, t3)[p])

        plsc.subcore_barrier()

        def _wb(i, carry):
            rr = s * RPT + i * ZR
            pltpu.sync_copy(acc_sh.at[pl.ds(rr, ZR)],
                            agg_hbm.at[c * 2 + p, pl.ds(rr, ZR)])
            return carry
        lax.fori_loop(0, RPT // ZR, _wb, 0)


def _sc1(src2d, dst2d, wrep, t0, t1, t2, t3):
    f = pl.kernel(
        _sc1_body,
        out_type=jax.ShapeDtypeStruct((4, N, 32), _f32),
        compiler_params=pltpu.CompilerParams(needs_layout_passes=False, use_tc_tiling_on_sc=False),
        mesh=plsc.VectorSubcoreMesh(core_axis_name="c", subcore_axis_name="s"),
        scratch_types=[
            pltpu.VMEM((SB, CH), _i32),
            pltpu.VMEM((SB, CH), _i32),
            pltpu.VMEM((CH, 32), _f32),
            pltpu.VMEM((CH, 32), _f32),
            pltpu.VMEM((CH, 32), _f32),
            pltpu.VMEM((CH, 32), _f32),
            pltpu.VMEM((CH, 16), _f32),
            pltpu.VMEM((CH, 16), _f32),
            pltpu.VMEM((CH, 16), _f32),
            pltpu.VMEM((CH, 16), _f32),
            pltpu.VMEM((CH, 32), _f32),
            pltpu.VMEM((CH, 32), _f32),
            pltpu.VMEM((CH, 32), _f32),
            pltpu.VMEM((CH, 32), _f32),
            pltpu.VMEM((ZR, 32), _f32),
            pltpu.VMEM_SHARED((N, 32), _f32),
            pltpu.SemaphoreType.DMA,
            pltpu.SemaphoreType.DMA,
            pltpu.SemaphoreType.DMA,
            pltpu.SemaphoreType.DMA,
            pltpu.SemaphoreType.DMA,
            pltpu.SemaphoreType.DMA,
            pltpu.SemaphoreType.DMA,
            pltpu.SemaphoreType.DMA,
        ],
    )
    return f(src2d, dst2d, wrep, t0, t1, t2, t3)


def _tc2_body(agg_ref, h0_ref, h1_ref, h2_ref, h3_ref,
              w2p_ref, wr2p_ref, b2p_ref, h2p_ref):
    agg = jnp.concatenate([agg_ref[0], agg_ref[1], agg_ref[2], agg_ref[3]],
                          axis=1)
    hc = jnp.concatenate([h0_ref[...], h1_ref[...], h2_ref[...], h3_ref[...]],
                         axis=1)
    h2p_ref[...] = jnp.maximum(_mm(agg, w2p_ref[...]) + b2p_ref[...]
                               + _mm(hc, wr2p_ref[...]), 0.0)


def _tc2(agg, h0, h1, h2, h3, w2p, wr2p, b2p):
    blk = 1024
    grid = (N // blk,)
    qspec = pl.BlockSpec((blk, 32), lambda i: (i, 0))
    return pl.pallas_call(
        _tc2_body,
        grid=grid,
        in_specs=[
            pl.BlockSpec((4, blk, 32), lambda i: (0, i, 0)),
            qspec, qspec, qspec, qspec,
            pl.BlockSpec((128, 16), lambda i: (0, 0)),
            pl.BlockSpec((128, 16), lambda i: (0, 0)),
            pl.BlockSpec((1, 16), lambda i: (0, 0)),
        ],
        out_specs=pl.BlockSpec((blk, 16), lambda i: (i, 0)),
        out_shape=jax.ShapeDtypeStruct((N, 16), _f32),
    )(agg, h0, h1, h2, h3, w2p, wr2p, b2p)


def _head_body(h2g_ref, scl_ref, sclg_ref, gf_ref,
               wg1_ref, bg1_ref, wg2_ref, bg2_ref, wg3_ref, bg3_ref,
               wo1a_ref, wo1b_ref, bo1_ref, wo2_ref, bo2_ref, out_ref):
    h2 = h2g_ref[...] * scl_ref[...]
    g = jnp.maximum(_mm(gf_ref[...], wg1_ref[...]) + bg1_ref[...], 0.0)
    g = jnp.maximum(_mm(g, wg2_ref[...]) + bg2_ref[...], 0.0)
    g = jnp.maximum(_mm(g, wg3_ref[...]) + bg3_ref[...], 0.0)
    g = g * sclg_ref[...]
    o1 = jnp.maximum(_mm(h2, wo1a_ref[...]) + _mm(g, wo1b_ref[...])
                     + bo1_ref[...], 0.0)
    o2 = _mm(o1, wo2_ref[...]) + bo2_ref[...]
    out_ref[...] = jax.nn.sigmoid(o2)


def _head(h2g, scl, sclg, gf,
          wg1, bg1, wg2, bg2, wg3, bg3, wo1a, wo1b, bo1, wo2, bo2):
    return pl.pallas_call(
        _head_body,
        out_shape=jax.ShapeDtypeStruct((B, 1), _f32),
    )(h2g, scl, sclg, gf,
      wg1, bg1, wg2, bg2, wg3, bg3, wo1a, wo1b, bo1, wo2, bo2)


def kernel(x, edge_index, edge_attr, globalFeats, isTrain,
           W_rel1, b_rel1, W_root1, W_rel2, b_rel2, W_root2,
           Wg1, bg1, Wg2, bg2, Wg3, bg3, Wo1, bo1, Wo2, bo2):
    src = jnp.asarray(edge_index[0], _i32).reshape(E // CH, CH)
    dst = jnp.asarray(edge_index[1], _i32).reshape(E // CH, CH)
    wrep = jnp.broadcast_to(jnp.asarray(edge_attr, _f32)[:, None], (E, 16))

    xq = [x[:, 32 * q:32 * (q + 1)] for q in range(4)]
    agg1 = _sc1(src, dst, wrep, *xq)
    h0, h1, h2, h3 = _tcmid(agg1, *xq, W_rel1, W_root1, b_rel1.reshape(1, 128))

    agg2 = _sc1(src, dst, wrep, h0, h1, h2, h3)
    pad = ((0, 0), (0, 12))
    w2p = jnp.pad(W_rel2, pad)
    wr2p = jnp.pad(W_root2, pad)
    b2p = jnp.pad(b_rel2, (0, 12)).reshape(1, 16)
    h2p = _tc2(agg2, h0, h1, h2, h3, w2p, wr2p, b2p)

    h2g = h2p.reshape(B, NPG * 16)

    d_cat = NPG * 4 + 16
    keep = jax.random.bernoulli(jax.random.key(42), 0.8, (B, d_cat))
    scale = jnp.where(jnp.asarray(isTrain),
                      jnp.where(keep, 1.0 / 0.8, 0.0),
                      1.0).astype(_f32)
    scl_emb = jnp.pad(scale[:, :NPG * 4].reshape(B, NPG, 4),
                      ((0, 0), (0, 0), (0, 12))).reshape(B, NPG * 16)
    scl_g = scale[:, NPG * 4:]

    wo1a = jnp.pad(Wo1[:NPG * 4].reshape(NPG, 4, 128),
                   ((0, 0), (0, 12), (0, 0))).reshape(NPG * 16, 128)
    wo1b = Wo1[NPG * 4:]

    return _head(h2g, scl_emb, scl_g, globalFeats,
                 Wg1, bg1.reshape(1, 8), Wg2, bg2.reshape(1, 8),
                 Wg3, bg3.reshape(1, 16), wo1a, wo1b, bo1.reshape(1, 128),
                 Wo2, bo2.reshape(1, 1))


# speedup vs baseline: 1.1552x; 1.0030x over previous
import functools

import jax
import jax.numpy as jnp
from jax import lax
from jax.experimental import pallas as pl
from jax.experimental.pallas import tpu as pltpu
from jax.experimental.pallas import tpu_sc as plsc

N = 27648
E = 442368
B = 512
NPG = 54
NC = 2
NS = 16
CH = 128
ZR = 216
RPT = N // NS

_f32 = jnp.float32
_i32 = jnp.int32


def _mm(a, b):
    return jnp.dot(a.astype(jnp.bfloat16), b.astype(jnp.bfloat16),
                   preferred_element_type=_f32)


def _tcmid_body(agg_ref, x0_ref, x1_ref, x2_ref, x3_ref,
                wrel_ref, wroot_ref, b1_ref,
                h0_ref, h1_ref, h2_ref, h3_ref):
    agg = jnp.concatenate([agg_ref[0], agg_ref[1], agg_ref[2], agg_ref[3]],
                          axis=1)
    xc = jnp.concatenate([x0_ref[...], x1_ref[...], x2_ref[...], x3_ref[...]],
                         axis=1)
    h = jnp.maximum(_mm(agg, wrel_ref[...]) + b1_ref[...]
                    + _mm(xc, wroot_ref[...]), 0.0)
    h0_ref[...] = h[:, 0:32]
    h1_ref[...] = h[:, 32:64]
    h2_ref[...] = h[:, 64:96]
    h3_ref[...] = h[:, 96:128]


def _tcmid(agg, x0, x1, x2, x3, wrel, wroot, b1):
    blk = 1024
    grid = (N // blk,)
    qspec = pl.BlockSpec((blk, 32), lambda i: (i, 0))
    return pl.pallas_call(
        _tcmid_body,
        grid=grid,
        in_specs=[
            pl.BlockSpec((4, blk, 32), lambda i: (0, i, 0)),
            qspec, qspec, qspec, qspec,
            pl.BlockSpec((128, 128), lambda i: (0, 0)),
            pl.BlockSpec((128, 128), lambda i: (0, 0)),
            pl.BlockSpec((1, 128), lambda i: (0, 0)),
        ],
        out_specs=[qspec, qspec, qspec, qspec],
        out_shape=[jax.ShapeDtypeStruct((N, 32), _f32)] * 4,
    )(agg, x0, x1, x2, x3, wrel, wroot, b1)


NB = 4
SB = 108


def _zero_acc(zbuf, acc_sh, s, qw, sem):
    def _zrow(r, carry):
        for q in range(qw):
            zbuf[r, pl.ds(q * 16, 16)] = jnp.zeros((16,), _f32)
        return carry
    lax.fori_loop(0, ZR, _zrow, 0)

    cps = [pltpu.async_copy(zbuf, acc_sh.at[pl.ds(s * RPT + i * ZR, ZR)], sem)
           for i in range(RPT // ZR)]
    for cp in cps:
        cp.wait()


def _edge_superchunk(row0, tbl_hbm, src2d, dst2d, wrep_hbm,
                     src_sv, dst_sv, rows, wreps, sbuf, acc_sh,
                     gsems, ssems, qw):
    pltpu.sync_copy(src2d.at[pl.ds(row0, SB)], src_sv)
    pltpu.sync_copy(dst2d.at[pl.ds(row0, SB)], dst_sv)
    for b in range(NB):
        pltpu.async_copy(tbl_hbm.at[src_sv.at[b]], rows[b], gsems[b])
        pltpu.async_copy(wrep_hbm.at[pl.ds((row0 + b) * CH, CH)],
                         wreps[b], gsems[b])

    def _round(r, carry):
        scats = []
        for b in range(NB):
            jl = r * NB + b
            pltpu.make_async_copy(tbl_hbm.at[src_sv.at[jl]],
                                  rows[b], gsems[b]).wait()
            pltpu.make_async_copy(wrep_hbm.at[pl.ds((row0 + jl) * CH, CH)],
                                  wreps[b], gsems[b]).wait()

            def _grp(g, cc):
                for u in range(16):
                    e = g * 16 + u
                    wb = wreps[b][e, :]
                    for q in range(qw):
                        sl = pl.ds(q * 16, 16)
                        sbuf[b][e, sl] = rows[b][e, sl] * wb
                return cc
            lax.fori_loop(0, CH // 16, _grp, 0)

            @pl.when(jl + NB < SB)
            def _():
                pltpu.async_copy(tbl_hbm.at[src_sv.at[jl + NB]],
                                 rows[b], gsems[b])
                pltpu.async_copy(wrep_hbm.at[pl.ds((row0 + jl + NB) * CH, CH)],
                                 wreps[b], gsems[b])
            scats.append(pltpu.async_copy(sbuf[b], acc_sh.at[dst_sv.at[jl]],
                                          ssems[b], add=True))
        for cp in scats:
            cp.wait()
        return carry
    lax.fori_loop(0, SB // NB, _round, 0)


def _sc1_body(src2d, dst2d, wrep_hbm, t0, t1, t2, t3, agg_hbm,
              src_sv, dst_sv,
              r0, r1, r2, r3, w0, w1, w2, w3, s0, s1, s2, s3, zbuf, acc_sh,
              g0, g1, g2, g3, q0, q1, q2, q3):
    c = lax.axis_index("c")
    s = lax.axis_index("s")
    rows, wreps = [r0, r1, r2, r3], [w0, w1, w2, w3]
    sbuf = [s0, s1, s2, s3]
    gsems, ssems = [g0, g1, g2, g3], [q0, q1, q2, q3]

    rpt = (E // CH) // NS

    for p in range(2):
        _zero_acc(zbuf, acc_sh, s, 2, g0)
        plsc.subcore_barrier()

        def _run(tbl_hbm):
            def _sc(k, carry):
                _edge_superchunk(s * rpt + k * SB, tbl_hbm,
                                 src2d, dst2d, wrep_hbm,
                                 src_sv, dst_sv, rows, wreps, sbuf, acc_sh,
                                 gsems, ssems, 2)
                return carry
            lax.fori_loop(0, rpt // SB, _sc, 0)

        @pl.when(c == 0)
        def _():
            _run((t0, t1)[p])

        @pl.when(c == 1)
        def _():
            _run((t2, t3)[p])

        plsc.subcore_barrier()

        cps = []
        for i in range(RPT // ZR):
            rr = s * RPT + i * ZR
            cps.append(pltpu.async_copy(
                acc_sh.at[pl.ds(rr, ZR)],
                agg_hbm.at[c * 2 + p, pl.ds(rr, ZR)], g0))
        for cp in cps:
            cp.wait()


def _sc1(src2d, dst2d, wrep, t0, t1, t2, t3):
    f = pl.kernel(
        _sc1_body,
        out_type=jax.ShapeDtypeStruct((4, N, 32), _f32),
        compiler_params=pltpu.CompilerParams(needs_layout_passes=False, use_tc_tiling_on_sc=False),
        mesh=plsc.VectorSubcoreMesh(core_axis_name="c", subcore_axis_name="s"),
        scratch_types=[
            pltpu.VMEM((SB, CH), _i32),
            pltpu.VMEM((SB, CH), _i32),
            pltpu.VMEM((CH, 32), _f32),
            pltpu.VMEM((CH, 32), _f32),
            pltpu.VMEM((CH, 32), _f32),
            pltpu.VMEM((CH, 32), _f32),
            pltpu.VMEM((CH, 16), _f32),
            pltpu.VMEM((CH, 16), _f32),
            pltpu.VMEM((CH, 16), _f32),
            pltpu.VMEM((CH, 16), _f32),
            pltpu.VMEM((CH, 32), _f32),
            pltpu.VMEM((CH, 32), _f32),
            pltpu.VMEM((CH, 32), _f32),
            pltpu.VMEM((CH, 32), _f32),
            pltpu.VMEM((ZR, 32), _f32),
            pltpu.VMEM_SHARED((N, 32), _f32),
            pltpu.SemaphoreType.DMA,
            pltpu.SemaphoreType.DMA,
            pltpu.SemaphoreType.DMA,
            pltpu.SemaphoreType.DMA,
            pltpu.SemaphoreType.DMA,
            pltpu.SemaphoreType.DMA,
            pltpu.SemaphoreType.DMA,
            pltpu.SemaphoreType.DMA,
        ],
    )
    return f(src2d, dst2d, wrep, t0, t1, t2, t3)


def _tc2_body(agg_ref, h0_ref, h1_ref, h2_ref, h3_ref,
              w2p_ref, wr2p_ref, b2p_ref, h2p_ref):
    agg = jnp.concatenate([agg_ref[0], agg_ref[1], agg_ref[2], agg_ref[3]],
                          axis=1)
    hc = jnp.concatenate([h0_ref[...], h1_ref[...], h2_ref[...], h3_ref[...]],
                         axis=1)
    h2p_ref[...] = jnp.maximum(_mm(agg, w2p_ref[...]) + b2p_ref[...]
                               + _mm(hc, wr2p_ref[...]), 0.0)


def _tc2(agg, h0, h1, h2, h3, w2p, wr2p, b2p):
    blk = 1024
    grid = (N // blk,)
    qspec = pl.BlockSpec((blk, 32), lambda i: (i, 0))
    return pl.pallas_call(
        _tc2_body,
        grid=grid,
        in_specs=[
            pl.BlockSpec((4, blk, 32), lambda i: (0, i, 0)),
            qspec, qspec, qspec, qspec,
            pl.BlockSpec((128, 16), lambda i: (0, 0)),
            pl.BlockSpec((128, 16), lambda i: (0, 0)),
            pl.BlockSpec((1, 16), lambda i: (0, 0)),
        ],
        out_specs=pl.BlockSpec((blk, 16), lambda i: (i, 0)),
        out_shape=jax.ShapeDtypeStruct((N, 16), _f32),
    )(agg, h0, h1, h2, h3, w2p, wr2p, b2p)


def _head_body(h2g_ref, scl_ref, sclg_ref, gf_ref,
               wg1_ref, bg1_ref, wg2_ref, bg2_ref, wg3_ref, bg3_ref,
               wo1a_ref, wo1b_ref, bo1_ref, wo2_ref, bo2_ref, out_ref):
    h2 = h2g_ref[...] * scl_ref[...]
    g = jnp.maximum(_mm(gf_ref[...], wg1_ref[...]) + bg1_ref[...], 0.0)
    g = jnp.maximum(_mm(g, wg2_ref[...]) + bg2_ref[...], 0.0)
    g = jnp.maximum(_mm(g, wg3_ref[...]) + bg3_ref[...], 0.0)
    g = g * sclg_ref[...]
    o1 = jnp.maximum(_mm(h2, wo1a_ref[...]) + _mm(g, wo1b_ref[...])
                     + bo1_ref[...], 0.0)
    o2 = _mm(o1, wo2_ref[...]) + bo2_ref[...]
    out_ref[...] = jax.nn.sigmoid(o2)


def _head(h2g, scl, sclg, gf,
          wg1, bg1, wg2, bg2, wg3, bg3, wo1a, wo1b, bo1, wo2, bo2):
    return pl.pallas_call(
        _head_body,
        out_shape=jax.ShapeDtypeStruct((B, 1), _f32),
    )(h2g, scl, sclg, gf,
      wg1, bg1, wg2, bg2, wg3, bg3, wo1a, wo1b, bo1, wo2, bo2)


def kernel(x, edge_index, edge_attr, globalFeats, isTrain,
           W_rel1, b_rel1, W_root1, W_rel2, b_rel2, W_root2,
           Wg1, bg1, Wg2, bg2, Wg3, bg3, Wo1, bo1, Wo2, bo2):
    src = jnp.asarray(edge_index[0], _i32).reshape(E // CH, CH)
    dst = jnp.asarray(edge_index[1], _i32).reshape(E // CH, CH)
    wrep = jnp.broadcast_to(jnp.asarray(edge_attr, _f32)[:, None], (E, 16))

    xq = [x[:, 32 * q:32 * (q + 1)] for q in range(4)]
    agg1 = _sc1(src, dst, wrep, *xq)
    h0, h1, h2, h3 = _tcmid(agg1, *xq, W_rel1, W_root1, b_rel1.reshape(1, 128))

    agg2 = _sc1(src, dst, wrep, h0, h1, h2, h3)
    pad = ((0, 0), (0, 12))
    w2p = jnp.pad(W_rel2, pad)
    wr2p = jnp.pad(W_root2, pad)
    b2p = jnp.pad(b_rel2, (0, 12)).reshape(1, 16)
    h2p = _tc2(agg2, h0, h1, h2, h3, w2p, wr2p, b2p)

    h2g = h2p.reshape(B, NPG * 16)

    d_cat = NPG * 4 + 16
    keep = jax.random.bernoulli(jax.random.key(42), 0.8, (B, d_cat))
    scale = jnp.where(jnp.asarray(isTrain),
                      jnp.where(keep, 1.0 / 0.8, 0.0),
                      1.0).astype(_f32)
    scl_emb = jnp.pad(scale[:, :NPG * 4].reshape(B, NPG, 4),
                      ((0, 0), (0, 0), (0, 12))).reshape(B, NPG * 16)
    scl_g = scale[:, NPG * 4:]

    wo1a = jnp.pad(Wo1[:NPG * 4].reshape(NPG, 4, 128),
                   ((0, 0), (0, 12), (0, 0))).reshape(NPG * 16, 128)
    wo1b = Wo1[NPG * 4:]

    return _head(h2g, scl_emb, scl_g, globalFeats,
                 Wg1, bg1.reshape(1, 8), Wg2, bg2.reshape(1, 8),
                 Wg3, bg3.reshape(1, 16), wo1a, wo1b, bo1.reshape(1, 128),
                 Wo2, bo2.reshape(1, 1))


# speedup vs baseline: 1.1567x; 1.0013x over previous
import jax
import jax.numpy as jnp
from jax import lax
from jax.experimental import pallas as pl
from jax.experimental.pallas import tpu as pltpu
from jax.experimental.pallas import tpu_sc as plsc

N = 27648
E = 442368
B = 512
NPG = 54
NC = 2
NS = 16
CH = 128
ZR = 216
RPT = N // NS

_f32 = jnp.float32
_i32 = jnp.int32


def _mm(a, b):
    return jnp.dot(a.astype(jnp.bfloat16), b.astype(jnp.bfloat16),
                   preferred_element_type=_f32)


def _tcmid_body(agg_ref, x0_ref, x1_ref, x2_ref, x3_ref,
                wrel_ref, wroot_ref, b1_ref,
                h0_ref, h1_ref, h2_ref, h3_ref):
    agg = jnp.concatenate([agg_ref[0], agg_ref[1], agg_ref[2], agg_ref[3]],
                          axis=1)
    xc = jnp.concatenate([x0_ref[...], x1_ref[...], x2_ref[...], x3_ref[...]],
                         axis=1)
    h = jnp.maximum(_mm(agg, wrel_ref[...]) + b1_ref[...]
                    + _mm(xc, wroot_ref[...]), 0.0)
    h0_ref[...] = h[:, 0:32]
    h1_ref[...] = h[:, 32:64]
    h2_ref[...] = h[:, 64:96]
    h3_ref[...] = h[:, 96:128]


def _tcmid(agg, x0, x1, x2, x3, wrel, wroot, b1):
    blk = 1024
    grid = (N // blk,)
    qspec = pl.BlockSpec((blk, 32), lambda i: (i, 0))
    return pl.pallas_call(
        _tcmid_body,
        grid=grid,
        in_specs=[
            pl.BlockSpec((4, blk, 32), lambda i: (0, i, 0)),
            qspec, qspec, qspec, qspec,
            pl.BlockSpec((128, 128), lambda i: (0, 0)),
            pl.BlockSpec((128, 128), lambda i: (0, 0)),
            pl.BlockSpec((1, 128), lambda i: (0, 0)),
        ],
        out_specs=[qspec, qspec, qspec, qspec],
        out_shape=[jax.ShapeDtypeStruct((N, 32), _f32)] * 4,
    )(agg, x0, x1, x2, x3, wrel, wroot, b1)


NB = 4
SB = 108


def _zero_acc(zbuf, acc_sh, s, qw, sem):
    def _zrow(r, carry):
        for q in range(qw):
            zbuf[r, pl.ds(q * 16, 16)] = jnp.zeros((16,), _f32)
        return carry
    lax.fori_loop(0, ZR, _zrow, 0)

    cps = [pltpu.async_copy(zbuf, acc_sh.at[pl.ds(s * RPT + i * ZR, ZR)], sem)
           for i in range(RPT // ZR)]
    for cp in cps:
        cp.wait()


def _edge_superchunk(row0, tbl_hbm, src2d, dst2d, wrep_hbm,
                     src_sv, dst_sv, rows, wreps, sbuf, acc_sh,
                     gsems, ssems, qw):
    pltpu.sync_copy(src2d.at[pl.ds(row0, SB)], src_sv)
    pltpu.sync_copy(dst2d.at[pl.ds(row0, SB)], dst_sv)
    for b in range(NB):
        pltpu.async_copy(tbl_hbm.at[src_sv.at[b]], rows[b], gsems[b])
        pltpu.async_copy(wrep_hbm.at[pl.ds((row0 + b) * CH, CH)],
                         wreps[b], gsems[b])

    def _round(r, carry):
        scats = []
        for b in range(NB):
            jl = r * NB + b
            pltpu.make_async_copy(tbl_hbm.at[src_sv.at[jl]],
                                  rows[b], gsems[b]).wait()
            pltpu.make_async_copy(wrep_hbm.at[pl.ds((row0 + jl) * CH, CH)],
                                  wreps[b], gsems[b]).wait()

            def _grp(g, cc):
                for u in range(16):
                    e = g * 16 + u
                    wb = wreps[b][e, :]
                    for q in range(qw):
                        sl = pl.ds(q * 16, 16)
                        sbuf[b][e, sl] = rows[b][e, sl] * wb
                return cc
            lax.fori_loop(0, CH // 16, _grp, 0)

            @pl.when(jl + NB < SB)
            def _():
                pltpu.async_copy(tbl_hbm.at[src_sv.at[jl + NB]],
                                 rows[b], gsems[b])
                pltpu.async_copy(wrep_hbm.at[pl.ds((row0 + jl + NB) * CH, CH)],
                                 wreps[b], gsems[b])
            scats.append(pltpu.async_copy(sbuf[b], acc_sh.at[dst_sv.at[jl]],
                                          ssems[b], add=True))
        for cp in scats:
            cp.wait()
        return carry
    lax.fori_loop(0, SB // NB, _round, 0)


def _sc1_body(src2d, dst2d, wrep_hbm, t0, t1, t2, t3, agg_hbm,
              src_sv, dst_sv,
              r0, r1, r2, r3, w0, w1, w2, w3, s0, s1, s2, s3, zbuf, acc_sh,
              g0, g1, g2, g3, q0, q1, q2, q3):
    c = lax.axis_index("c")
    s = lax.axis_index("s")
    rows, wreps = [r0, r1, r2, r3], [w0, w1, w2, w3]
    sbuf = [s0, s1, s2, s3]
    gsems, ssems = [g0, g1, g2, g3], [q0, q1, q2, q3]

    rpt = (E // CH) // NS

    for p in range(2):
        _zero_acc(zbuf, acc_sh, s, 2, g0)
        plsc.subcore_barrier()

        def _run(tbl_hbm):
            def _sc(k, carry):
                _edge_superchunk(s * rpt + k * SB, tbl_hbm,
                                 src2d, dst2d, wrep_hbm,
                                 src_sv, dst_sv, rows, wreps, sbuf, acc_sh,
                                 gsems, ssems, 2)
                return carry
            lax.fori_loop(0, rpt // SB, _sc, 0)

        @pl.when(c == 0)
        def _():
            _run((t0, t1)[p])

        @pl.when(c == 1)
        def _():
            _run((t2, t3)[p])

        plsc.subcore_barrier()

        cps = []
        for i in range(RPT // ZR):
            rr = s * RPT + i * ZR
            cps.append(pltpu.async_copy(
                acc_sh.at[pl.ds(rr, ZR)],
                agg_hbm.at[c * 2 + p, pl.ds(rr, ZR)], g0))
        for cp in cps:
            cp.wait()


def _sc1(src2d, dst2d, wrep, t0, t1, t2, t3):
    f = pl.kernel(
        _sc1_body,
        out_type=jax.ShapeDtypeStruct((4, N, 32), _f32),
        compiler_params=pltpu.CompilerParams(needs_layout_passes=False, use_tc_tiling_on_sc=False),
        mesh=plsc.VectorSubcoreMesh(core_axis_name="c", subcore_axis_name="s"),
        scratch_types=[
            pltpu.VMEM((SB, CH), _i32),
            pltpu.VMEM((SB, CH), _i32),
            pltpu.VMEM((CH, 32), _f32),
            pltpu.VMEM((CH, 32), _f32),
            pltpu.VMEM((CH, 32), _f32),
            pltpu.VMEM((CH, 32), _f32),
            pltpu.VMEM((CH, 16), _f32),
            pltpu.VMEM((CH, 16), _f32),
            pltpu.VMEM((CH, 16), _f32),
            pltpu.VMEM((CH, 16), _f32),
            pltpu.VMEM((CH, 32), _f32),
            pltpu.VMEM((CH, 32), _f32),
            pltpu.VMEM((CH, 32), _f32),
            pltpu.VMEM((CH, 32), _f32),
            pltpu.VMEM((ZR, 32), _f32),
            pltpu.VMEM_SHARED((N, 32), _f32),
            pltpu.SemaphoreType.DMA,
            pltpu.SemaphoreType.DMA,
            pltpu.SemaphoreType.DMA,
            pltpu.SemaphoreType.DMA,
            pltpu.SemaphoreType.DMA,
            pltpu.SemaphoreType.DMA,
            pltpu.SemaphoreType.DMA,
            pltpu.SemaphoreType.DMA,
        ],
    )
    return f(src2d, dst2d, wrep, t0, t1, t2, t3)


def _tc2_body(agg_ref, h0_ref, h1_ref, h2_ref, h3_ref,
              w2p_ref, wr2p_ref, b2p_ref, h2p_ref):
    agg = jnp.concatenate([agg_ref[0], agg_ref[1], agg_ref[2], agg_ref[3]],
                          axis=1)
    hc = jnp.concatenate([h0_ref[...], h1_ref[...], h2_ref[...], h3_ref[...]],
                         axis=1)
    h2p_ref[...] = jnp.maximum(_mm(agg, w2p_ref[...]) + b2p_ref[...]
                               + _mm(hc, wr2p_ref[...]), 0.0)


def _tc2(agg, h0, h1, h2, h3, w2p, wr2p, b2p):
    blk = 1024
    grid = (N // blk,)
    qspec = pl.BlockSpec((blk, 32), lambda i: (i, 0))
    return pl.pallas_call(
        _tc2_body,
        grid=grid,
        in_specs=[
            pl.BlockSpec((4, blk, 32), lambda i: (0, i, 0)),
            qspec, qspec, qspec, qspec,
            pl.BlockSpec((128, 16), lambda i: (0, 0)),
            pl.BlockSpec((128, 16), lambda i: (0, 0)),
            pl.BlockSpec((1, 16), lambda i: (0, 0)),
        ],
        out_specs=pl.BlockSpec((blk, 16), lambda i: (i, 0)),
        out_shape=jax.ShapeDtypeStruct((N, 16), _f32),
    )(agg, h0, h1, h2, h3, w2p, wr2p, b2p)


def _head_body(h2g_ref, scl_ref, sclg_ref, gf_ref,
               wg1_ref, bg1_ref, wg2_ref, bg2_ref, wg3_ref, bg3_ref,
               wo1a_ref, wo1b_ref, bo1_ref, wo2_ref, bo2_ref, out_ref):
    h2 = h2g_ref[...] * scl_ref[...]
    g = jnp.maximum(_mm(gf_ref[...], wg1_ref[...]) + bg1_ref[...], 0.0)
    g = jnp.maximum(_mm(g, wg2_ref[...]) + bg2_ref[...], 0.0)
    g = jnp.maximum(_mm(g, wg3_ref[...]) + bg3_ref[...], 0.0)
    g = g * sclg_ref[...]
    o1 = jnp.maximum(_mm(h2, wo1a_ref[...]) + _mm(g, wo1b_ref[...])
                     + bo1_ref[...], 0.0)
    o2 = _mm(o1, wo2_ref[...]) + bo2_ref[...]
    out_ref[...] = jax.nn.sigmoid(o2)


def _head(h2g, scl, sclg, gf,
          wg1, bg1, wg2, bg2, wg3, bg3, wo1a, wo1b, bo1, wo2, bo2):
    return pl.pallas_call(
        _head_body,
        out_shape=jax.ShapeDtypeStruct((B, 1), _f32),
    )(h2g, scl, sclg, gf,
      wg1, bg1, wg2, bg2, wg3, bg3, wo1a, wo1b, bo1, wo2, bo2)


def kernel(x, edge_index, edge_attr, globalFeats, isTrain,
           W_rel1, b_rel1, W_root1, W_rel2, b_rel2, W_root2,
           Wg1, bg1, Wg2, bg2, Wg3, bg3, Wo1, bo1, Wo2, bo2):
    src = jnp.asarray(edge_index[0], _i32).reshape(E // CH, CH)
    dst = jnp.asarray(edge_index[1], _i32).reshape(E // CH, CH)
    wrep = jnp.broadcast_to(jnp.asarray(edge_attr, _f32)[:, None], (E, 16))

    xq = [x[:, 32 * q:32 * (q + 1)] for q in range(4)]
    agg1 = _sc1(src, dst, wrep, *xq)
    h0, h1, h2, h3 = _tcmid(agg1, *xq, W_rel1, W_root1, b_rel1.reshape(1, 128))

    agg2 = _sc1(src, dst, wrep, h0, h1, h2, h3)
    pad = ((0, 0), (0, 12))
    w2p = jnp.pad(W_rel2, pad)
    wr2p = jnp.pad(W_root2, pad)
    b2p = jnp.pad(b_rel2, (0, 12)).reshape(1, 16)
    h2p = _tc2(agg2, h0, h1, h2, h3, w2p, wr2p, b2p)

    h2g = h2p.reshape(B, NPG * 16)

    d_cat = NPG * 4 + 16
    keep = jax.random.bernoulli(jax.random.key(42), 0.8, (B, d_cat))
    scale = jnp.where(jnp.asarray(isTrain),
                      jnp.where(keep, 1.0 / 0.8, 0.0),
                      1.0).astype(_f32)
    scl_emb = jnp.pad(scale[:, :NPG * 4].reshape(B, NPG, 4),
                      ((0, 0), (0, 0), (0, 12))).reshape(B, NPG * 16)
    scl_g = scale[:, NPG * 4:]

    wo1a = jnp.pad(Wo1[:NPG * 4].reshape(NPG, 4, 128),
                   ((0, 0), (0, 12), (0, 0))).reshape(NPG * 16, 128)
    wo1b = Wo1[NPG * 4:]

    return _head(h2g, scl_emb, scl_g, globalFeats,
                 Wg1, bg1.reshape(1, 8), Wg2, bg2.reshape(1, 8),
                 Wg3, bg3.reshape(1, 16), wo1a, wo1b, bo1.reshape(1, 128),
                 Wo2, bo2.reshape(1, 1))
